# baseline (speedup 1.0000x reference)
import jax
import jax.numpy as jnp
from jax import lax
from jax.experimental import pallas as pl
from jax.experimental.pallas import tpu as pltpu
from jax.experimental.pallas import tpu_sc as plsc

N = 25000
D = 128
K = 7
KP = 8
NC, NS = 2, 16
NW = NC * NS
ROWS_PER_CHUNK = 16
IDX_PER_CHUNK = ROWS_PER_CHUNK * KP
CHUNKS_PER_WORKER = 49
NBUF = 4


def _sc_body(x_hbm, idx_hbm, out_hbm, idx_f, gbuf, obuf, gsem, osem):
    wid = lax.axis_index("s") * NC + lax.axis_index("c")
    rows_per_worker = CHUNKS_PER_WORKER * ROWS_PER_CHUNK
    base_row = jnp.minimum(wid * rows_per_worker, N - rows_per_worker)
    idx_per_worker = CHUNKS_PER_WORKER * IDX_PER_CHUNK
    pltpu.sync_copy(idx_hbm.at[pl.ds(base_row * KP, idx_per_worker)], idx_f)

    def gather_copy(c, slot):
        return pltpu.make_async_copy(
            x_hbm.at[idx_f.at[pl.ds(c * IDX_PER_CHUNK, IDX_PER_CHUNK)]],
            gbuf.at[slot],
            gsem.at[slot],
        )

    def out_slice(c):
        return out_hbm.at[pl.ds(base_row + c * ROWS_PER_CHUNK, ROWS_PER_CHUNK), :]

    for c in range(NBUF - 1):
        gather_copy(c, c).start()

    def chunk_body(c, carry):
        slot = lax.rem(c, NBUF)

        @pl.when(c + NBUF - 1 < CHUNKS_PER_WORKER)
        def _():
            gather_copy(c + NBUF - 1, lax.rem(c + NBUF - 1, NBUF)).start()

        gather_copy(c, slot).wait()

        @pl.when(c >= NBUF)
        def _():
            pltpu.make_async_copy(obuf.at[slot], out_slice(c), osem.at[slot]).wait()

        def row_body(i, carry2):
            for g in range(D // 16):
                s = pl.ds(g * 16, 16)
                m = gbuf[slot, i * KP, s]
                for k in range(1, K):
                    m = jnp.maximum(m, gbuf[slot, i * KP + k, s])
                obuf[slot, i, s] = m
            return carry2

        lax.fori_loop(0, ROWS_PER_CHUNK, row_body, 0)
        pltpu.async_copy(obuf.at[slot], out_slice(c), osem.at[slot])
        return carry

    lax.fori_loop(0, CHUNKS_PER_WORKER, chunk_body, 0)
    for slot in range(NBUF):
        pltpu.make_async_copy(obuf.at[slot], out_slice(0), osem.at[slot]).wait()


@jax.jit
def _hex_pool(x, idx_flat):
    mesh = plsc.VectorSubcoreMesh(
        core_axis_name="c", subcore_axis_name="s", num_cores=NC, num_subcores=NS
    )
    return pl.kernel(
        _sc_body,
        out_type=jax.ShapeDtypeStruct((N, D), jnp.float32),
        mesh=mesh,
        scratch_types=[
            pltpu.VMEM((CHUNKS_PER_WORKER * IDX_PER_CHUNK,), jnp.int32),
            pltpu.VMEM((NBUF, IDX_PER_CHUNK, D), jnp.float32),
            pltpu.VMEM((NBUF, ROWS_PER_CHUNK, D), jnp.float32),
            pltpu.SemaphoreType.DMA((NBUF,)),
            pltpu.SemaphoreType.DMA((NBUF,)),
        ],
    )(x, idx_flat)


def kernel(x, neigh_indices):
    ni = neigh_indices.astype(jnp.int32)
    ni8 = jnp.concatenate([ni, jnp.zeros((N, 1), jnp.int32)], axis=1)
    return _hex_pool(x, ni8.reshape(-1))


# speedup vs baseline: 12.1093x; 12.1093x over previous
import jax
import jax.numpy as jnp
from jax import lax
from jax.experimental import pallas as pl
from jax.experimental.pallas import tpu as pltpu
from jax.experimental.pallas import tpu_sc as plsc

N = 25000
D = 128
K = 7
KP = 8
NC, NS = 2, 16
NW = NC * NS
ROWS_PER_CHUNK = 16
IDX_PER_CHUNK = ROWS_PER_CHUNK * KP
CHUNKS_PER_WORKER = 49
NBUF = 4


def _sc_body(x_hbm, idx_hbm, out_hbm, idx_f, gbuf, obuf, gsem, osem):
    wid = lax.axis_index("s") * NC + lax.axis_index("c")
    rows_per_worker = CHUNKS_PER_WORKER * ROWS_PER_CHUNK
    base_row = jnp.minimum(wid * rows_per_worker, N - rows_per_worker)
    idx_per_worker = CHUNKS_PER_WORKER * IDX_PER_CHUNK
    pltpu.sync_copy(idx_hbm.at[pl.ds(base_row * KP, idx_per_worker)], idx_f)

    def gather_copy(c, slot):
        return pltpu.make_async_copy(
            x_hbm.at[idx_f.at[pl.ds(c * IDX_PER_CHUNK, IDX_PER_CHUNK)]],
            gbuf.at[slot],
            gsem.at[slot],
        )

    def out_slice(c):
        return out_hbm.at[pl.ds(base_row + c * ROWS_PER_CHUNK, ROWS_PER_CHUNK), :]

    for c in range(NBUF - 1):
        gather_copy(c, c).start()

    def chunk_body(c, carry):
        slot = lax.rem(c, NBUF)

        @pl.when(c + NBUF - 1 < CHUNKS_PER_WORKER)
        def _():
            gather_copy(c + NBUF - 1, lax.rem(c + NBUF - 1, NBUF)).start()

        gather_copy(c, slot).wait()

        @pl.when(c >= NBUF)
        def _():
            pltpu.make_async_copy(obuf.at[slot], out_slice(c), osem.at[slot]).wait()

        def row_body(i, carry2):
            for g in range(D // 16):
                s = pl.ds(g * 16, 16)
                m = gbuf[slot, i * KP, s]
                for k in range(1, K):
                    m = jnp.maximum(m, gbuf[slot, i * KP + k, s])
                obuf[slot, i, s] = m
            return carry2

        lax.fori_loop(0, ROWS_PER_CHUNK, row_body, 0)
        pltpu.async_copy(obuf.at[slot], out_slice(c), osem.at[slot])
        return carry

    lax.fori_loop(0, CHUNKS_PER_WORKER, chunk_body, 0)
    for slot in range(NBUF):
        pltpu.make_async_copy(obuf.at[slot], out_slice(0), osem.at[slot]).wait()


@jax.jit
def _hex_pool(x, idx_flat):
    mesh = plsc.VectorSubcoreMesh(
        core_axis_name="c", subcore_axis_name="s", num_cores=NC, num_subcores=NS
    )
    return pl.kernel(
        _sc_body,
        out_type=jax.ShapeDtypeStruct((N, D), jnp.float32),
        mesh=mesh,
        scratch_types=[
            pltpu.VMEM((CHUNKS_PER_WORKER * IDX_PER_CHUNK,), jnp.int32),
            pltpu.VMEM((NBUF, IDX_PER_CHUNK, D), jnp.float32),
            pltpu.VMEM((NBUF, ROWS_PER_CHUNK, D), jnp.float32),
            pltpu.SemaphoreType.DMA((NBUF,)),
            pltpu.SemaphoreType.DMA((NBUF,)),
        ],
    )(x, idx_flat)


def kernel(x, neigh_indices):
    ni = neigh_indices.astype(jnp.int32)
    rowid = lax.broadcasted_iota(jnp.int32, (N, 1), 0)
    ni8 = jnp.concatenate([ni, rowid], axis=1)
    return _hex_pool(x, ni8.reshape(-1))


# speedup vs baseline: 12.1904x; 1.0067x over previous
import jax
import jax.numpy as jnp
from jax import lax
from jax.experimental import pallas as pl
from jax.experimental.pallas import tpu as pltpu
from jax.experimental.pallas import tpu_sc as plsc

N = 25000
D = 128
K = 7
NC, NS = 2, 16
NW = NC * NS
ROWS_PER_CHUNK = 16
IDX_PER_CHUNK = ROWS_PER_CHUNK * K
CHUNKS_PER_WORKER = 49
NBUF = 4
SROWS = 112
NBLOCK = CHUNKS_PER_WORKER * ROWS_PER_CHUNK // SROWS


def _sc_body(x_hbm, idx_hbm, out_hbm, sbuf, idx_f, gbuf, obuf, ssem, gsem, osem):
    wid = lax.axis_index("s") * NC + lax.axis_index("c")
    rows_per_worker = CHUNKS_PER_WORKER * ROWS_PER_CHUNK
    base_row = jnp.minimum(wid * rows_per_worker, N - rows_per_worker)

    def stage_copy(b, slot):
        return pltpu.make_async_copy(
            idx_hbm.at[pl.ds(base_row + b * SROWS, SROWS), :],
            sbuf.at[slot],
            ssem.at[slot],
        )

    stage_copy(0, 0).start()
    for b in range(NBLOCK):
        if b + 1 < NBLOCK:
            stage_copy(b + 1, (b + 1) % 2).start()
        stage_copy(b, b % 2).wait()

        def flat_body(cc, carry, b=b):
            slots = jnp.full((16,), b % 2, jnp.int32)
            rows = cc * ROWS_PER_CHUNK + lax.iota(jnp.int32, 16)
            for k in range(K):
                cols = jnp.full((16,), k, jnp.int32)
                idx_f[
                    pl.ds(b * SROWS * K + cc * IDX_PER_CHUNK + k * 16, 16)
                ] = plsc.load_gather(sbuf, [slots, rows, cols])
            return carry

        lax.fori_loop(0, SROWS // ROWS_PER_CHUNK, flat_body, 0)

    def gather_copy(c, slot):
        return pltpu.make_async_copy(
            x_hbm.at[idx_f.at[pl.ds(c * IDX_PER_CHUNK, IDX_PER_CHUNK)]],
            gbuf.at[slot],
            gsem.at[slot],
        )

    def out_slice(c):
        return out_hbm.at[pl.ds(base_row + c * ROWS_PER_CHUNK, ROWS_PER_CHUNK), :]

    for c in range(NBUF - 1):
        gather_copy(c, c).start()

    def chunk_body(c, carry):
        slot = lax.rem(c, NBUF)

        @pl.when(c + NBUF - 1 < CHUNKS_PER_WORKER)
        def _():
            gather_copy(c + NBUF - 1, lax.rem(c + NBUF - 1, NBUF)).start()

        gather_copy(c, slot).wait()

        @pl.when(c >= NBUF)
        def _():
            pltpu.make_async_copy(obuf.at[slot], out_slice(c), osem.at[slot]).wait()

        def row_body(i, carry2):
            for g in range(D // 16):
                s = pl.ds(g * 16, 16)
                m = gbuf[slot, i, s]
                for k in range(1, K):
                    m = jnp.maximum(m, gbuf[slot, i + k * ROWS_PER_CHUNK, s])
                obuf[slot, i, s] = m
            return carry2

        lax.fori_loop(0, ROWS_PER_CHUNK, row_body, 0)
        pltpu.async_copy(obuf.at[slot], out_slice(c), osem.at[slot])
        return carry

    lax.fori_loop(0, CHUNKS_PER_WORKER, chunk_body, 0)
    for slot in range(NBUF):
        pltpu.make_async_copy(obuf.at[slot], out_slice(0), osem.at[slot]).wait()


@jax.jit
def _hex_pool(x, ni):
    mesh = plsc.VectorSubcoreMesh(
        core_axis_name="c", subcore_axis_name="s", num_cores=NC, num_subcores=NS
    )
    return pl.kernel(
        _sc_body,
        out_type=jax.ShapeDtypeStruct((N, D), jnp.float32),
        mesh=mesh,
        compiler_params=pltpu.CompilerParams(needs_layout_passes=False),
        scratch_types=[
            pltpu.VMEM((2, SROWS, K), jnp.int32),
            pltpu.VMEM((CHUNKS_PER_WORKER * IDX_PER_CHUNK,), jnp.int32),
            pltpu.VMEM((NBUF, IDX_PER_CHUNK, D), jnp.float32),
            pltpu.VMEM((NBUF, ROWS_PER_CHUNK, D), jnp.float32),
            pltpu.SemaphoreType.DMA((2,)),
            pltpu.SemaphoreType.DMA((NBUF,)),
            pltpu.SemaphoreType.DMA((NBUF,)),
        ],
    )(x, ni)


def kernel(x, neigh_indices):
    return _hex_pool(x, neigh_indices.astype(jnp.int32))


# speedup vs baseline: 12.2025x; 1.0010x over previous
import jax
import jax.numpy as jnp
from jax import lax
from jax.experimental import pallas as pl
from jax.experimental.pallas import tpu as pltpu
from jax.experimental.pallas import tpu_sc as plsc

N = 25000
D = 128
K = 7
NC, NS = 2, 16
NW = NC * NS
ROWS_PER_CHUNK = 16
IDX_PER_CHUNK = ROWS_PER_CHUNK * K
CHUNKS_PER_WORKER = 49
NBUF = 4
SROWS = 112
NBLOCK = CHUNKS_PER_WORKER * ROWS_PER_CHUNK // SROWS


def _sc_body(x_hbm, idx_hbm, out_hbm, sbuf, idx_f, gbuf, obuf, ssem, gsem, osem):
    wid = lax.axis_index("s") * NC + lax.axis_index("c")
    rows_per_worker = CHUNKS_PER_WORKER * ROWS_PER_CHUNK
    base_row = jnp.minimum(wid * rows_per_worker, N - rows_per_worker)

    def stage_copy(b, slot):
        return pltpu.make_async_copy(
            idx_hbm.at[pl.ds(base_row + b * SROWS, SROWS), :],
            sbuf.at[slot],
            ssem.at[slot],
        )

    stage_copy(0, 0).start()
    for b in range(NBLOCK):
        if b + 1 < NBLOCK:
            stage_copy(b + 1, (b + 1) % 2).start()
        stage_copy(b, b % 2).wait()

        def flat_body(cc, carry, b=b):
            slots = jnp.full((16,), b % 2, jnp.int32)
            rows = cc * ROWS_PER_CHUNK + lax.iota(jnp.int32, 16)
            for k in range(K):
                cols = jnp.full((16,), k, jnp.int32)
                idx_f[
                    pl.ds(b * SROWS * K + cc * IDX_PER_CHUNK + k * 16, 16)
                ] = plsc.load_gather(sbuf, [slots, rows, cols])
            return carry

        lax.fori_loop(0, SROWS // ROWS_PER_CHUNK, flat_body, 0)

    def gather_copy(c, slot):
        return pltpu.make_async_copy(
            x_hbm.at[idx_f.at[pl.ds(c * IDX_PER_CHUNK, IDX_PER_CHUNK)]],
            gbuf.at[slot],
            gsem.at[slot],
        )

    def out_slice(c):
        return out_hbm.at[pl.ds(base_row + c * ROWS_PER_CHUNK, ROWS_PER_CHUNK), :]

    for c in range(NBUF - 1):
        gather_copy(c, c).start()

    def chunk_body(c, carry):
        slot = lax.rem(c, NBUF)

        @pl.when(c + NBUF - 1 < CHUNKS_PER_WORKER)
        def _():
            gather_copy(c + NBUF - 1, lax.rem(c + NBUF - 1, NBUF)).start()

        gather_copy(c, slot).wait()

        @pl.when(c >= NBUF)
        def _():
            pltpu.make_async_copy(obuf.at[slot], out_slice(c), osem.at[slot]).wait()

        def row_body(i, carry2):
            for g in range(D // 16):
                s = pl.ds(g * 16, 16)
                m = gbuf[slot, i, s]
                for k in range(1, K):
                    m = jnp.maximum(m, gbuf[slot, i + k * ROWS_PER_CHUNK, s])
                obuf[slot, i, s] = m
            return carry2

        lax.fori_loop(0, ROWS_PER_CHUNK, row_body, 0)
        pltpu.async_copy(obuf.at[slot], out_slice(c), osem.at[slot])
        return carry

    lax.fori_loop(0, CHUNKS_PER_WORKER, chunk_body, 0)
    for slot in range(NBUF):
        pltpu.make_async_copy(obuf.at[slot], out_slice(0), osem.at[slot]).wait()


@jax.jit
def _hex_pool(x, ni):
    mesh = plsc.VectorSubcoreMesh(
        core_axis_name="c", subcore_axis_name="s", num_cores=NC, num_subcores=NS
    )
    return pl.kernel(
        _sc_body,
        out_type=jax.ShapeDtypeStruct((N, D), jnp.float32),
        mesh=mesh,
        compiler_params=pltpu.CompilerParams(
            needs_layout_passes=False, use_tc_tiling_on_sc=True
        ),
        scratch_types=[
            pltpu.VMEM((2, SROWS, K), jnp.int32),
            pltpu.VMEM((CHUNKS_PER_WORKER * IDX_PER_CHUNK,), jnp.int32),
            pltpu.VMEM((NBUF, IDX_PER_CHUNK, D), jnp.float32),
            pltpu.VMEM((NBUF, ROWS_PER_CHUNK, D), jnp.float32),
            pltpu.SemaphoreType.DMA((2,)),
            pltpu.SemaphoreType.DMA((NBUF,)),
            pltpu.SemaphoreType.DMA((NBUF,)),
        ],
    )(x, ni)


def kernel(x, neigh_indices):
    return _hex_pool(x, neigh_indices.astype(jnp.int32))


# speedup vs baseline: 12.7554x; 1.0453x over previous
import jax
import jax.numpy as jnp
from jax import lax
from jax.experimental import pallas as pl
from jax.experimental.pallas import tpu as pltpu
from jax.experimental.pallas import tpu_sc as plsc

N = 25000
D = 128
K = 7
NC, NS = 2, 16
NW = NC * NS
ROWS_PER_CHUNK = 16
IDX_PER_CHUNK = ROWS_PER_CHUNK * K
CHUNKS_PER_WORKER = 49
NBUF = 4
NSTAGE = 8


def _sc_body(x_hbm, idx_hbm, out_hbm, sbuf, idx_f, gbuf, obuf, ssem, gsem, osem):
    wid = lax.axis_index("s") * NC + lax.axis_index("c")
    rows_per_worker = CHUNKS_PER_WORKER * ROWS_PER_CHUNK
    base_row = jnp.minimum(wid * rows_per_worker, N - rows_per_worker)

    def stage_copy(c, slot):
        return pltpu.make_async_copy(
            idx_hbm.at[pl.ds(base_row + c * ROWS_PER_CHUNK, ROWS_PER_CHUNK), :],
            sbuf.at[slot],
            ssem.at[slot],
        )

    def flatten(c, sslot, islot):
        slots = jnp.full((16,), sslot, jnp.int32)
        rows = lax.iota(jnp.int32, 16)
        for k in range(K):
            cols = jnp.full((16,), k, jnp.int32)
            idx_f[islot, pl.ds(k * 16, 16)] = plsc.load_gather(
                sbuf, [slots, rows, cols]
            )

    def gather_copy(slot):
        return pltpu.make_async_copy(
            x_hbm.at[idx_f.at[slot]],
            gbuf.at[slot],
            gsem.at[slot],
        )

    def out_slice(c):
        return out_hbm.at[pl.ds(base_row + c * ROWS_PER_CHUNK, ROWS_PER_CHUNK), :]

    for j in range(NBUF + 2):
        stage_copy(j, j % NSTAGE).start()
    for j in range(NBUF - 1):
        stage_copy(j, j % NSTAGE).wait()
        flatten(j, j % NSTAGE, j)
        gather_copy(j).start()

    def chunk_body(c, carry):
        slot = lax.rem(c, NBUF)

        @pl.when(c + NBUF + 2 < CHUNKS_PER_WORKER)
        def _():
            stage_copy(c + NBUF + 2, lax.rem(c + NBUF + 2, NSTAGE)).start()

        @pl.when(c + NBUF - 1 < CHUNKS_PER_WORKER)
        def _():
            cn = c + NBUF - 1
            sslot = lax.rem(cn, NSTAGE)
            nslot = lax.rem(cn, NBUF)
            stage_copy(cn, sslot).wait()
            flatten(cn, sslot, nslot)
            gather_copy(nslot).start()

        gather_copy(slot).wait()

        @pl.when(c >= NBUF)
        def _():
            pltpu.make_async_copy(obuf.at[slot], out_slice(c), osem.at[slot]).wait()

        def row_body(i, carry2):
            for g in range(D // 16):
                s = pl.ds(g * 16, 16)
                m = gbuf[slot, i, s]
                for k in range(1, K):
                    m = jnp.maximum(m, gbuf[slot, i + k * ROWS_PER_CHUNK, s])
                obuf[slot, i, s] = m
            return carry2

        lax.fori_loop(0, ROWS_PER_CHUNK, row_body, 0)
        pltpu.async_copy(obuf.at[slot], out_slice(c), osem.at[slot])
        return carry

    lax.fori_loop(0, CHUNKS_PER_WORKER, chunk_body, 0)
    for slot in range(NBUF):
        pltpu.make_async_copy(obuf.at[slot], out_slice(0), osem.at[slot]).wait()


@jax.jit
def _hex_pool(x, ni):
    mesh = plsc.VectorSubcoreMesh(
        core_axis_name="c", subcore_axis_name="s", num_cores=NC, num_subcores=NS
    )
    return pl.kernel(
        _sc_body,
        out_type=jax.ShapeDtypeStruct((N, D), jnp.float32),
        mesh=mesh,
        compiler_params=pltpu.CompilerParams(
            needs_layout_passes=False, use_tc_tiling_on_sc=True
        ),
        scratch_types=[
            pltpu.VMEM((NSTAGE, ROWS_PER_CHUNK, K), jnp.int32),
            pltpu.VMEM((NBUF, IDX_PER_CHUNK), jnp.int32),
            pltpu.VMEM((NBUF, IDX_PER_CHUNK, D), jnp.float32),
            pltpu.VMEM((NBUF, ROWS_PER_CHUNK, D), jnp.float32),
            pltpu.SemaphoreType.DMA((NSTAGE,)),
            pltpu.SemaphoreType.DMA((NBUF,)),
            pltpu.SemaphoreType.DMA((NBUF,)),
        ],
    )(x, ni)


def kernel(x, neigh_indices):
    return _hex_pool(x, neigh_indices.astype(jnp.int32))
